# baseline (speedup 1.0000x reference)
import functools

import jax
import jax.numpy as jnp
from jax import lax
from jax.experimental import pallas as pl
from jax.experimental.pallas import tpu as pltpu
from jax.experimental.pallas import tpu_sc as plsc

_BATCH = 16384
_DIM = 64
_MARGIN = 2.0

_NC = 2
_NS = 16
_L = 16
_NW = _NC * _NS
_BPW = _BATCH // _NW
_R = 128
_NCHUNK = _BPW // _R


def _sqrt16(x):
    i = plsc.bitcast(x, jnp.int32)
    y = plsc.bitcast(jnp.int32(0x5F3759DF) - (i >> 1), jnp.float32)
    for _ in range(3):
        y = y * (1.5 - 0.5 * x * y * y)
    return x * y


def _tec_body(hs_hbm, ts_hbm, rs_hbm, ent_hbm, rel_hbm, nv_hbm, out_hbm,
              h_idx, t_idx, r_idx, h_rows, t_rows, r_rows, n_rows, out_v, sem):
    wid = lax.axis_index("s") * _NC + lax.axis_index("c")
    base = wid * _BPW
    lane = lax.broadcasted_iota(jnp.int32, (_L,), 0)

    for c in range(_NCHUNK):
        cbase = base + c * _R
        pltpu.sync_copy(hs_hbm.at[pl.ds(cbase, _R)], h_idx)
        pltpu.sync_copy(ts_hbm.at[pl.ds(cbase, _R)], t_idx)
        pltpu.sync_copy(rs_hbm.at[pl.ds(cbase, _R)], r_idx)

        ch = pltpu.async_copy(ent_hbm.at[h_idx], h_rows, sem)
        ct = pltpu.async_copy(ent_hbm.at[t_idx], t_rows, sem)
        cr = pltpu.async_copy(rel_hbm.at[r_idx], r_rows, sem)
        cn = pltpu.async_copy(nv_hbm.at[r_idx], n_rows, sem)
        ch.wait()
        ct.wait()
        cr.wait()
        cn.wait()

        for g in range(_R // _L):
            row_idx = lane + (g * _L)
            zero = jnp.zeros((_L,), jnp.float32)

            def body(j, acc):
                a, b, cc, d, e, f = acc
                col = jnp.full((_L,), j, jnp.int32)
                hj = plsc.load_gather(h_rows, [row_idx, col])
                tj = plsc.load_gather(t_rows, [row_idx, col])
                rj = plsc.load_gather(r_rows, [row_idx, col])
                nj = plsc.load_gather(n_rows, [row_idx, col])
                u = hj - tj
                a = a + u * u
                b = b + u * nj
                cc = cc + u * rj
                d = d + rj * nj
                e = e + nj * nj
                f = f + rj * rj
                return (a, b, cc, d, e, f)

            a, b, cc, d, e, f = lax.fori_loop(
                0, _DIM, body, (zero, zero, zero, zero, zero, zero))
            total = a + b * b * e + f - 2.0 * b * b + 2.0 * cc - 2.0 * b * d
            score = _MARGIN - _sqrt16(total)
            out_v[pl.ds(c * _R + g * _L, _L)] = score

    pltpu.sync_copy(out_v, out_hbm.at[pl.ds(base, _BPW)])


_mesh = plsc.VectorSubcoreMesh(core_axis_name="c", subcore_axis_name="s")

_sc_call = functools.partial(
    pl.kernel,
    mesh=_mesh,
    compiler_params=pltpu.CompilerParams(
        needs_layout_passes=False, use_tc_tiling_on_sc=False),
    out_type=jax.ShapeDtypeStruct((_BATCH,), jnp.float32),
    scratch_types=[
        pltpu.VMEM((_R,), jnp.int32),
        pltpu.VMEM((_R,), jnp.int32),
        pltpu.VMEM((_R,), jnp.int32),
        pltpu.VMEM((_R, _DIM), jnp.float32),
        pltpu.VMEM((_R, _DIM), jnp.float32),
        pltpu.VMEM((_R, _DIM), jnp.float32),
        pltpu.VMEM((_R, _DIM), jnp.float32),
        pltpu.VMEM((_BPW,), jnp.float32),
        pltpu.SemaphoreType.DMA,
    ],
)(_tec_body)


@jax.jit
def kernel(hs, rs, ts, ent_embs, rel_embs, norm_vector):
    scores = _sc_call(hs.astype(jnp.int32), ts.astype(jnp.int32),
                      rs.astype(jnp.int32), ent_embs, rel_embs, norm_vector)
    return scores.reshape(_BATCH, 1)


# speedup vs baseline: 1.7538x; 1.7538x over previous
import functools

import jax
import jax.numpy as jnp
from jax import lax
from jax.experimental import pallas as pl
from jax.experimental.pallas import tpu as pltpu
from jax.experimental.pallas import tpu_sc as plsc

_BATCH = 16384
_DIM = 64
_MARGIN = 2.0

_NC = 2
_NS = 16
_L = 16
_NW = _NC * _NS
_BPW = _BATCH // _NW
_CH = 16
_NCHUNK = _BPW // _CH


def _sqrt16(x):
    i = plsc.bitcast(x, jnp.int32)
    y = plsc.bitcast(jnp.int32(0x5F3759DF) - (i >> 1), jnp.float32)
    for _ in range(3):
        y = y * (1.5 - 0.5 * x * y * y)
    return x * y


def _rsum(v):
    return lax.reduce_sum_p.bind(v, axes=(0,))


def _tec_body(hs_hbm, ts_hbm, rs_hbm, ent_hbm, rel_hbm, nv_hbm, out_hbm,
              hv, tv, rv, h_rows, t_rows, r_rows, n_rows, out_v, sem):
    wid = lax.axis_index("s") * _NC + lax.axis_index("c")
    base = wid * _BPW
    lane = lax.broadcasted_iota(jnp.int32, (_L,), 0)

    pltpu.sync_copy(hs_hbm.at[pl.ds(base, _BPW)], hv)
    pltpu.sync_copy(ts_hbm.at[pl.ds(base, _BPW)], tv)
    pltpu.sync_copy(rs_hbm.at[pl.ds(base, _BPW)], rv)

    def chunk(c, _):
        hvec = hv[pl.ds(c * _CH, _CH)]
        tvec = tv[pl.ds(c * _CH, _CH)]
        rvec = rv[pl.ds(c * _CH, _CH)]
        copies = []
        for i in range(_CH):
            copies.append(pltpu.async_copy(
                ent_hbm.at[hvec[i]], h_rows.at[i], sem))
            copies.append(pltpu.async_copy(
                ent_hbm.at[tvec[i]], t_rows.at[i], sem))
            copies.append(pltpu.async_copy(
                rel_hbm.at[rvec[i]], r_rows.at[i], sem))
            copies.append(pltpu.async_copy(
                nv_hbm.at[rvec[i]], n_rows.at[i], sem))
        for cp in copies:
            cp.wait()

        total_vec = jnp.zeros((_L,), jnp.float32)
        for i in range(_CH):
            pa = pb = pc = pd = pe = pf = jnp.zeros((_L,), jnp.float32)
            for k in range(_DIM // _L):
                sl = pl.ds(k * _L, _L)
                hk = h_rows[i, sl]
                tk = t_rows[i, sl]
                rk = r_rows[i, sl]
                nk = n_rows[i, sl]
                u = hk - tk
                pa = pa + u * u
                pb = pb + u * nk
                pc = pc + u * rk
                pd = pd + rk * nk
                pe = pe + nk * nk
                pf = pf + rk * rk
            a = _rsum(pa)
            b = _rsum(pb)
            cdot = _rsum(pc)
            d = _rsum(pd)
            e = _rsum(pe)
            f = _rsum(pf)
            tot = a + b * b * e + f - 2.0 * b * b + 2.0 * cdot - 2.0 * b * d
            total_vec = jnp.where(lane == i, tot, total_vec)
        out_v[pl.ds(c * _CH, _CH)] = _MARGIN - _sqrt16(total_vec)
        return _

    lax.fori_loop(0, _NCHUNK, chunk, 0)
    pltpu.sync_copy(out_v, out_hbm.at[pl.ds(base, _BPW)])


_mesh = plsc.VectorSubcoreMesh(core_axis_name="c", subcore_axis_name="s")

_sc_call = functools.partial(
    pl.kernel,
    mesh=_mesh,
    compiler_params=pltpu.CompilerParams(
        needs_layout_passes=False, use_tc_tiling_on_sc=True),
    out_type=jax.ShapeDtypeStruct((_BATCH,), jnp.float32),
    scratch_types=[
        pltpu.VMEM((_BPW,), jnp.int32),
        pltpu.VMEM((_BPW,), jnp.int32),
        pltpu.VMEM((_BPW,), jnp.int32),
        pltpu.VMEM((_CH, _DIM), jnp.float32),
        pltpu.VMEM((_CH, _DIM), jnp.float32),
        pltpu.VMEM((_CH, _DIM), jnp.float32),
        pltpu.VMEM((_CH, _DIM), jnp.float32),
        pltpu.VMEM((_BPW,), jnp.float32),
        pltpu.SemaphoreType.DMA,
    ],
)(_tec_body)


@jax.jit
def kernel(hs, rs, ts, ent_embs, rel_embs, norm_vector):
    scores = _sc_call(hs.astype(jnp.int32), ts.astype(jnp.int32),
                      rs.astype(jnp.int32), ent_embs, rel_embs, norm_vector)
    return scores.reshape(_BATCH, 1)


# speedup vs baseline: 1.8405x; 1.0494x over previous
import functools

import jax
import jax.numpy as jnp
from jax import lax
from jax.experimental import pallas as pl
from jax.experimental.pallas import tpu as pltpu
from jax.experimental.pallas import tpu_sc as plsc

_BATCH = 16384
_DIM = 64
_MARGIN = 2.0

_NC = 2
_NS = 16
_L = 16
_NW = _NC * _NS
_BPW = _BATCH // _NW
_CH = 16
_NCHUNK = _BPW // _CH


def _sqrt16(x):
    i = plsc.bitcast(x, jnp.int32)
    y = plsc.bitcast(jnp.int32(0x5F3759DF) - (i >> 1), jnp.float32)
    for _ in range(3):
        y = y * (1.5 - 0.5 * x * y * y)
    return x * y


def _rsum(v):
    return lax.reduce_sum_p.bind(v, axes=(0,))


def _tec_body(hs_hbm, ts_hbm, rs_hbm, ent_hbm, rn_hbm, out_hbm,
              hv, tv, rv, h_rows, t_rows, rn_rows, out_v, sem_a, sem_b):
    wid = lax.axis_index("s") * _NC + lax.axis_index("c")
    base = wid * _BPW
    lane = lax.broadcasted_iota(jnp.int32, (_L,), 0)

    pltpu.sync_copy(hs_hbm.at[pl.ds(base, _BPW)], hv)
    pltpu.sync_copy(ts_hbm.at[pl.ds(base, _BPW)], tv)
    pltpu.sync_copy(rs_hbm.at[pl.ds(base, _BPW)], rv)

    def fire(c, slot, sem):
        hvec = hv[pl.ds(c * _CH, _CH)]
        tvec = tv[pl.ds(c * _CH, _CH)]
        rvec = rv[pl.ds(c * _CH, _CH)]
        for i in range(_CH):
            pltpu.async_copy(ent_hbm.at[hvec[i]], h_rows.at[slot, i], sem)
            pltpu.async_copy(ent_hbm.at[tvec[i]], t_rows.at[slot, i], sem)
        pltpu.async_copy(rn_hbm.at[rvec], rn_rows.at[slot], sem)

    def drain(slot, sem):
        for i in range(_CH):
            pltpu.make_async_copy(ent_hbm.at[0], h_rows.at[slot, i], sem).wait()
            pltpu.make_async_copy(ent_hbm.at[0], t_rows.at[slot, i], sem).wait()
        pltpu.make_async_copy(
            rn_hbm.at[pl.ds(0, _CH)], rn_rows.at[slot], sem).wait()

    def compute(c, slot):
        total_vec = jnp.zeros((_L,), jnp.float32)
        for i in range(_CH):
            pa = pb = pc = pd = pe = pf = jnp.zeros((_L,), jnp.float32)
            for k in range(_DIM // _L):
                sl = pl.ds(k * _L, _L)
                hk = h_rows[slot, i, sl]
                tk = t_rows[slot, i, sl]
                rk = rn_rows[slot, i, sl]
                nk = rn_rows[slot, i, pl.ds(_DIM + k * _L, _L)]
                u = hk - tk
                pa = pa + u * u
                pb = pb + u * nk
                pc = pc + u * rk
                pd = pd + rk * nk
                pe = pe + nk * nk
                pf = pf + rk * rk
            a = _rsum(pa)
            b = _rsum(pb)
            cdot = _rsum(pc)
            d = _rsum(pd)
            e = _rsum(pe)
            f = _rsum(pf)
            tot = a + b * b * e + f - 2.0 * b * b + 2.0 * cdot - 2.0 * b * d
            total_vec = jnp.where(lane == i, tot, total_vec)
        out_v[pl.ds(c * _CH, _CH)] = _MARGIN - _sqrt16(total_vec)

    fire(0, 0, sem_a)

    def body2(m, carry):
        c0 = 2 * m
        c1 = c0 + 1
        fire(c1, 1, sem_b)
        drain(0, sem_a)
        compute(c0, 0)
        pl.when(c1 + 1 < _NCHUNK)(lambda: fire(c1 + 1, 0, sem_a))
        drain(1, sem_b)
        compute(c1, 1)
        return carry

    lax.fori_loop(0, _NCHUNK // 2, body2, 0)
    pltpu.sync_copy(out_v, out_hbm.at[pl.ds(base, _BPW)])


_mesh = plsc.VectorSubcoreMesh(core_axis_name="c", subcore_axis_name="s")

_sc_call = functools.partial(
    pl.kernel,
    mesh=_mesh,
    compiler_params=pltpu.CompilerParams(
        needs_layout_passes=False, use_tc_tiling_on_sc=True),
    out_type=jax.ShapeDtypeStruct((_BATCH,), jnp.float32),
    scratch_types=[
        pltpu.VMEM((_BPW,), jnp.int32),
        pltpu.VMEM((_BPW,), jnp.int32),
        pltpu.VMEM((_BPW,), jnp.int32),
        pltpu.VMEM((2, _CH, _DIM), jnp.float32),
        pltpu.VMEM((2, _CH, _DIM), jnp.float32),
        pltpu.VMEM((2, _CH, 2 * _DIM), jnp.float32),
        pltpu.VMEM((_BPW,), jnp.float32),
        pltpu.SemaphoreType.DMA,
        pltpu.SemaphoreType.DMA,
    ],
)(_tec_body)


@jax.jit
def kernel(hs, rs, ts, ent_embs, rel_embs, norm_vector):
    rn = jnp.concatenate([rel_embs, norm_vector], axis=1)
    scores = _sc_call(hs.astype(jnp.int32), ts.astype(jnp.int32),
                      rs.astype(jnp.int32), ent_embs, rn)
    return scores.reshape(_BATCH, 1)
